# baseline (speedup 1.0000x reference)
import functools

import jax
import jax.numpy as jnp
from jax import lax
from jax.experimental import pallas as pl
from jax.experimental.pallas import tpu as pltpu
from jax.experimental.pallas import tpu_sc as plsc

_BATCH = 4096
_SEQ = 200
_D = 64
_TOT = _BATCH * _SEQ
_NC, _NS = 2, 16
_NW = _NC * _NS
_PER_W = _TOT // _NW
_CH = 128
_NCHUNK = _PER_W // _CH
_NBUF = 6


def _make_gather():
    mesh = plsc.VectorSubcoreMesh(core_axis_name="c", subcore_axis_name="s")

    @functools.partial(
        pl.kernel,
        mesh=mesh,
        compiler_params=pltpu.CompilerParams(use_tc_tiling_on_sc=False),
        out_type=jax.ShapeDtypeStruct((_TOT, 2 * _D), jnp.float32),
        scratch_types=[
            pltpu.VMEM((_NCHUNK, _CH), jnp.int32),
            pltpu.VMEM((_NBUF, _CH, _D), jnp.float32),
            pltpu.SemaphoreType.DMA((_NBUF,)),
            pltpu.SemaphoreType.DMA((_NBUF,)),
        ],
    )
    def gather_kernel(idx_hbm, table_hbm, out_hbm, idx_v, rows_v, gsem, ssem):
        wid = lax.axis_index("s") * _NC + lax.axis_index("c")
        pltpu.sync_copy(idx_hbm.at[pl.ds(wid * _NCHUNK, _NCHUNK)], idx_v)
        out_base = wid * _PER_W

        def start_gather(g, slot):
            pltpu.async_copy(
                table_hbm.at[idx_v.at[g]], rows_v.at[slot], gsem.at[slot]
            )

        def gather_desc(slot):
            return pltpu.make_async_copy(
                table_hbm.at[idx_v.at[0]], rows_v.at[slot], gsem.at[slot]
            )

        def start_store(h, slot):
            pltpu.async_copy(
                rows_v.at[slot],
                out_hbm.at[pl.ds(out_base + h * _CH, _CH), pl.ds(0, _D)],
                ssem.at[slot],
            )

        def store_desc(slot):
            return pltpu.make_async_copy(
                rows_v.at[slot],
                out_hbm.at[pl.ds(out_base, _CH), pl.ds(0, _D)],
                ssem.at[slot],
            )

        _LAG = _NBUF - 1

        def body(g, carry):
            slot = lax.rem(g, _NBUF)

            @pl.when(g >= _NBUF)
            def _():
                store_desc(slot).wait()

            start_gather(g, slot)

            @pl.when(g >= _LAG)
            def _():
                h = g - _LAG
                hslot = lax.rem(h, _NBUF)
                gather_desc(hslot).wait()
                start_store(h, hslot)

            return carry

        lax.fori_loop(0, _NCHUNK, body, 0, unroll=2)

        for h in range(_NCHUNK - _LAG, _NCHUNK):
            slot = h % _NBUF
            gather_desc(slot).wait()
            start_store(h, slot)
        for h in range(_NCHUNK - _NBUF, _NCHUNK):
            store_desc(h % _NBUF).wait()

    return gather_kernel


_gather = _make_gather()


def kernel(input_ids, word_embeddings):
    ids = input_ids.reshape(_TOT // _CH, _CH).astype(jnp.int32)
    padded = _gather(ids, word_embeddings)
    return padded[:, :_D].reshape(_BATCH, _SEQ, _D)
